# baseline (speedup 1.0000x reference)
import functools

import jax
import jax.numpy as jnp
from jax import lax
from jax.experimental import pallas as pl
from jax.experimental.pallas import tpu as pltpu
from jax.experimental.pallas import tpu_sc as plsc

BATCH = 16384
SEQ = 35
N_BEAKERS = SEQ // 5
IDS_PER_ROW = 4 * N_BEAKERS
D = 64
VOCAB = 7

QUADS_PER_ROW = IDS_PER_ROW // 4
DP = 4 * D
PVOCAB = VOCAB ** 4

NUM_CORES = 2
NUM_SUBCORES = 16
NW = NUM_CORES * NUM_SUBCORES

IDX_MINOR = 128
ROWS_TOTAL = BATCH * QUADS_PER_ROW
IDX_ROWS = ROWS_TOTAL // IDX_MINOR
IDX_ROWS_PER_W = IDX_ROWS // NW
CHUNK_IDX_ROWS = 1
CHUNK_ROWS = CHUNK_IDX_ROWS * IDX_MINOR
N_CHUNKS = IDX_ROWS_PER_W // CHUNK_IDX_ROWS


def _make_sc_gather():
    mesh = plsc.VectorSubcoreMesh(core_axis_name="c", subcore_axis_name="s")

    @functools.partial(
        pl.kernel,
        mesh=mesh,
        out_type=jax.ShapeDtypeStruct((ROWS_TOTAL, DP // 128, 128), jnp.float32),
        scratch_types=[
            pltpu.VMEM_SHARED((PVOCAB, DP // 128, 128), jnp.float32),
            pltpu.VMEM((IDX_ROWS_PER_W, IDX_MINOR), jnp.int32),
            pltpu.VMEM((2, CHUNK_ROWS, DP // 128, 128), jnp.float32),
            pltpu.SemaphoreType.DMA,
            pltpu.SemaphoreType.DMA,
        ],
    )
    def sc_gather(table_hbm, idx_hbm, out_hbm, table_s, idx_v, rows_v, sem_g, sem_o):
        wid = lax.axis_index("s") * NUM_CORES + lax.axis_index("c")
        idx_base = wid * IDX_ROWS_PER_W

        @pl.when(lax.axis_index("s") == 0)
        def _():
            pltpu.sync_copy(table_hbm, table_s)

        pltpu.sync_copy(idx_hbm.at[wid], idx_v)
        plsc.subcore_barrier()

        def gather_descs(ci, buf):
            return [
                pltpu.make_async_copy(
                    table_s.at[idx_v.at[ci * CHUNK_IDX_ROWS + j]],
                    buf.at[pl.ds(j * IDX_MINOR, IDX_MINOR)],
                    sem_g,
                )
                for j in range(CHUNK_IDX_ROWS)
            ]

        def out_desc(ci, buf):
            out_row = (idx_base + ci * CHUNK_IDX_ROWS) * IDX_MINOR
            return pltpu.make_async_copy(
                buf, out_hbm.at[pl.ds(out_row, CHUNK_ROWS)], sem_o)

        for d in gather_descs(0, rows_v.at[0]):
            d.start()

        def chunk_body(ci, carry):
            buf = rows_v.at[ci % 2]
            nbuf = rows_v.at[(ci + 1) % 2]

            @pl.when(ci >= 1)
            def _():
                out_desc(ci - 1, nbuf).wait()

            @pl.when(ci + 1 < N_CHUNKS)
            def _():
                for d in gather_descs(ci + 1, nbuf):
                    d.start()

            for d in gather_descs(ci, buf):
                d.wait()
            out_desc(ci, buf).start()
            return carry

        lax.fori_loop(0, N_CHUNKS, chunk_body, 0)
        out_desc(N_CHUNKS - 1, rows_v.at[(N_CHUNKS - 1) % 2]).wait()

    return sc_gather


_sc_gather = _make_sc_gather()


def kernel(X, color_table, pos_table):
    del pos_table
    ids = X.reshape(BATCH, N_BEAKERS, 5)[:, :, 1:5].astype(jnp.int32)
    ids = ids.reshape(BATCH, QUADS_PER_ROW, 4)
    quad_ids = ((ids[:, :, 0] * VOCAB + ids[:, :, 1]) * VOCAB
                + ids[:, :, 2]) * VOCAB + ids[:, :, 3]
    idx = quad_ids.reshape(NW, IDX_ROWS_PER_W, IDX_MINOR)
    V = VOCAB
    qtable = jnp.concatenate(
        [jnp.repeat(color_table, V ** 3, axis=0),
         jnp.tile(jnp.repeat(color_table, V ** 2, axis=0), (V, 1)),
         jnp.tile(jnp.repeat(color_table, V, axis=0), (V ** 2, 1)),
         jnp.tile(color_table, (V ** 3, 1))], axis=1)
    rows = _sc_gather(qtable.reshape(PVOCAB, DP // 128, 128), idx)
    return rows.reshape(BATCH, IDS_PER_ROW * D)


# speedup vs baseline: 2.1357x; 2.1357x over previous
import functools

import jax
import jax.numpy as jnp
from jax import lax
from jax.experimental import pallas as pl
from jax.experimental.pallas import tpu as pltpu
from jax.experimental.pallas import tpu_sc as plsc

BATCH = 16384
SEQ = 35
N_BEAKERS = SEQ // 5
IDS_PER_ROW = 4 * N_BEAKERS
D = 64
VOCAB = 7

PAIRS_PER_ROW = IDS_PER_ROW // 2
DP = 2 * D
PVOCAB = VOCAB * VOCAB

NUM_CORES = 2
NUM_SUBCORES = 16
NW = NUM_CORES * NUM_SUBCORES

IDX_MINOR = 128
ROWS_TOTAL = BATCH * PAIRS_PER_ROW
IDX_ROWS = ROWS_TOTAL // IDX_MINOR
IDX_ROWS_PER_W = IDX_ROWS // NW
CHUNK_IDX_ROWS = 2
CHUNK_ROWS = CHUNK_IDX_ROWS * IDX_MINOR
N_CHUNKS = IDX_ROWS_PER_W // CHUNK_IDX_ROWS


def _make_sc_gather():
    mesh = plsc.VectorSubcoreMesh(core_axis_name="c", subcore_axis_name="s")

    @functools.partial(
        pl.kernel,
        mesh=mesh,
        out_type=jax.ShapeDtypeStruct((ROWS_TOTAL, DP), jnp.float32),
        scratch_types=[
            pltpu.VMEM_SHARED((PVOCAB, DP), jnp.float32),
            pltpu.VMEM((IDX_ROWS_PER_W, IDX_MINOR), jnp.int32),
            pltpu.VMEM((2, CHUNK_ROWS, DP), jnp.float32),
            pltpu.SemaphoreType.DMA,
            pltpu.SemaphoreType.DMA,
        ],
    )
    def sc_gather(table_hbm, idx_hbm, out_hbm, table_s, idx_v, rows_v, sem_g, sem_o):
        wid = lax.axis_index("s") * NUM_CORES + lax.axis_index("c")
        idx_base = wid * IDX_ROWS_PER_W

        @pl.when(lax.axis_index("s") == 0)
        def _():
            pltpu.sync_copy(table_hbm, table_s)

        pltpu.sync_copy(idx_hbm.at[pl.ds(idx_base, IDX_ROWS_PER_W)], idx_v)
        plsc.subcore_barrier()

        def gather_descs(ci, buf):
            return [
                pltpu.make_async_copy(
                    table_s.at[idx_v.at[ci * CHUNK_IDX_ROWS + j]],
                    buf.at[pl.ds(j * IDX_MINOR, IDX_MINOR)],
                    sem_g,
                )
                for j in range(CHUNK_IDX_ROWS)
            ]

        def out_desc(ci, buf):
            out_row = (idx_base + ci * CHUNK_IDX_ROWS) * IDX_MINOR
            return pltpu.make_async_copy(
                buf, out_hbm.at[pl.ds(out_row, CHUNK_ROWS)], sem_o)

        for d in gather_descs(0, rows_v.at[0]):
            d.start()

        def chunk_body(ci, carry):
            buf = rows_v.at[ci % 2]
            nbuf = rows_v.at[(ci + 1) % 2]

            @pl.when(ci >= 1)
            def _():
                out_desc(ci - 1, nbuf).wait()

            @pl.when(ci + 1 < N_CHUNKS)
            def _():
                for d in gather_descs(ci + 1, nbuf):
                    d.start()

            for d in gather_descs(ci, buf):
                d.wait()
            out_desc(ci, buf).start()
            return carry

        lax.fori_loop(0, N_CHUNKS, chunk_body, 0)
        out_desc(N_CHUNKS - 1, rows_v.at[(N_CHUNKS - 1) % 2]).wait()

    return sc_gather


_sc_gather = _make_sc_gather()


def kernel(X, color_table, pos_table):
    del pos_table
    ids = X.reshape(BATCH, N_BEAKERS, 5)[:, :, 1:5].astype(jnp.int32)
    ids = ids.reshape(BATCH, PAIRS_PER_ROW, 2)
    pair_ids = ids[:, :, 0] * VOCAB + ids[:, :, 1]
    idx = pair_ids.reshape(IDX_ROWS, IDX_MINOR)
    ptable = jnp.concatenate(
        [jnp.repeat(color_table, VOCAB, axis=0),
         jnp.tile(color_table, (VOCAB, 1))], axis=1)
    rows = _sc_gather(ptable, idx)
    return rows.reshape(BATCH, IDS_PER_ROW * D)


# speedup vs baseline: 5.1807x; 2.4258x over previous
import functools

import jax
import jax.numpy as jnp
from jax import lax
from jax.experimental import pallas as pl
from jax.experimental.pallas import tpu as pltpu
from jax.experimental.pallas import tpu_sc as plsc

BATCH = 16384
SEQ = 35
N_BEAKERS = SEQ // 5
IDS_PER_ROW = 4 * N_BEAKERS
D = 64
VOCAB = 7

PAIRS_PER_ROW = IDS_PER_ROW // 2
DP = 2 * D
PVOCAB = VOCAB * VOCAB

NUM_CORES = 2
NUM_SUBCORES = 16
NW = NUM_CORES * NUM_SUBCORES

IDX_MINOR = 8 * PAIRS_PER_ROW
ROWS_TOTAL = BATCH * PAIRS_PER_ROW
IDX_ROWS = ROWS_TOTAL // IDX_MINOR
IDX_ROWS_PER_W = IDX_ROWS // NW
CHUNK_IDX_ROWS = 2
CHUNK_ROWS = CHUNK_IDX_ROWS * IDX_MINOR
CHUNK_B = CHUNK_ROWS // PAIRS_PER_ROW
N_CHUNKS = IDX_ROWS_PER_W // CHUNK_IDX_ROWS


def _make_sc_gather():
    mesh = plsc.VectorSubcoreMesh(core_axis_name="c", subcore_axis_name="s")

    @functools.partial(
        pl.kernel,
        mesh=mesh,
        out_type=jax.ShapeDtypeStruct((BATCH, IDS_PER_ROW * D), jnp.float32),
        scratch_types=[
            pltpu.VMEM_SHARED((PVOCAB, DP), jnp.float32),
            pltpu.VMEM((IDX_ROWS_PER_W, IDX_MINOR), jnp.int32),
            pltpu.VMEM((2, CHUNK_ROWS, DP), jnp.float32),
            pltpu.SemaphoreType.DMA,
            pltpu.SemaphoreType.DMA,
        ],
    )
    def sc_gather(table_hbm, idx_hbm, out_hbm, table_s, idx_v, rows_v, sem_g, sem_o):
        wid = lax.axis_index("s") * NUM_CORES + lax.axis_index("c")
        idx_base = wid * IDX_ROWS_PER_W

        @pl.when(lax.axis_index("s") == 0)
        def _():
            pltpu.sync_copy(table_hbm, table_s)

        pltpu.sync_copy(idx_hbm.at[pl.ds(idx_base, IDX_ROWS_PER_W)], idx_v)
        plsc.subcore_barrier()

        def gather_descs(ci, buf):
            return [
                pltpu.make_async_copy(
                    table_s.at[idx_v.at[ci * CHUNK_IDX_ROWS + j]],
                    buf.at[pl.ds(j * IDX_MINOR, IDX_MINOR)],
                    sem_g,
                )
                for j in range(CHUNK_IDX_ROWS)
            ]

        def out_desc(ci, buf):
            out_row = (idx_base + ci * CHUNK_IDX_ROWS) * (IDX_MINOR // PAIRS_PER_ROW)
            return pltpu.make_async_copy(
                buf.reshape(CHUNK_B, IDS_PER_ROW * D),
                out_hbm.at[pl.ds(out_row, CHUNK_B)], sem_o)

        for d in gather_descs(0, rows_v.at[0]):
            d.start()

        def chunk_body(ci, carry):
            buf = rows_v.at[ci % 2]
            nbuf = rows_v.at[(ci + 1) % 2]

            @pl.when(ci >= 1)
            def _():
                out_desc(ci - 1, nbuf).wait()

            @pl.when(ci + 1 < N_CHUNKS)
            def _():
                for d in gather_descs(ci + 1, nbuf):
                    d.start()

            for d in gather_descs(ci, buf):
                d.wait()
            out_desc(ci, buf).start()
            return carry

        lax.fori_loop(0, N_CHUNKS, chunk_body, 0)
        out_desc(N_CHUNKS - 1, rows_v.at[(N_CHUNKS - 1) % 2]).wait()

    return sc_gather


_sc_gather = _make_sc_gather()


def kernel(X, color_table, pos_table):
    del pos_table
    ids = X.reshape(BATCH, N_BEAKERS, 5)[:, :, 1:5].astype(jnp.int32)
    ids = ids.reshape(BATCH, PAIRS_PER_ROW, 2)
    pair_ids = ids[:, :, 0] * VOCAB + ids[:, :, 1]
    idx = pair_ids.reshape(IDX_ROWS, IDX_MINOR)
    ptable = jnp.concatenate(
        [jnp.repeat(color_table, VOCAB, axis=0),
         jnp.tile(color_table, (VOCAB, 1))], axis=1)
    return _sc_gather(ptable, idx)


# speedup vs baseline: 5.5387x; 1.0691x over previous
import functools

import numpy as np

import jax
import jax.numpy as jnp
from jax import lax
from jax.experimental import pallas as pl
from jax.experimental.pallas import tpu as pltpu
from jax.experimental.pallas import tpu_sc as plsc

BATCH = 16384
SEQ = 35
N_BEAKERS = SEQ // 5
IDS_PER_ROW = 4 * N_BEAKERS
D = 64
VOCAB = 7

PAIRS_PER_ROW = IDS_PER_ROW // 2
DP = 2 * D
PVOCAB = VOCAB * VOCAB

NUM_CORES = 2
NUM_SUBCORES = 16
NW = NUM_CORES * NUM_SUBCORES
LANES = 16

B_PER_W = BATCH // NW
PAIRS_PER_W = B_PER_W * PAIRS_PER_ROW
ID_STEPS = PAIRS_PER_W // LANES

IDX_MINOR = 8 * PAIRS_PER_ROW
IDX_ROWS_PER_W = PAIRS_PER_W // IDX_MINOR
CHUNK_IDX_ROWS = 2
CHUNK_ROWS = CHUNK_IDX_ROWS * IDX_MINOR
CHUNK_B = CHUNK_ROWS // PAIRS_PER_ROW
N_CHUNKS = IDX_ROWS_PER_W // CHUNK_IDX_ROWS


def _make_sc_gather():
    mesh = plsc.VectorSubcoreMesh(core_axis_name="c", subcore_axis_name="s")

    @functools.partial(
        pl.kernel,
        mesh=mesh,
        compiler_params=pltpu.CompilerParams(needs_layout_passes=False),
        out_type=jax.ShapeDtypeStruct((BATCH, IDS_PER_ROW * D), jnp.float32),
        scratch_types=[
            pltpu.VMEM_SHARED((PVOCAB, DP), jnp.float32),
            pltpu.VMEM((B_PER_W, SEQ), jnp.int32),
            pltpu.VMEM((PAIRS_PER_W,), jnp.int32),
            pltpu.VMEM((2, CHUNK_ROWS, DP), jnp.float32),
            pltpu.SemaphoreType.DMA,
            pltpu.SemaphoreType.DMA,
        ],
    )
    def sc_gather(table_hbm, x_hbm, out_hbm, table_s, x_v, idx_v, rows_v,
                  sem_g, sem_o):
        wid = lax.axis_index("s") * NUM_CORES + lax.axis_index("c")
        b_base = wid * B_PER_W

        @pl.when(lax.axis_index("s") == 0)
        def _():
            pltpu.sync_copy(table_hbm, table_s)

        pltpu.sync_copy(x_hbm.at[pl.ds(b_base, B_PER_W)], x_v)

        def id_step(r, carry):
            lane = lax.iota(jnp.int32, LANES)
            for k in range(7):
                g = k * LANES + lane
                q = (g * 2341) >> 15
                p = g - PAIRS_PER_ROW * q
                c0 = 5 * (p >> 1) + 1 + 2 * (p & 1)
                b = r * 8 + q
                a = plsc.load_gather(x_v, [b, c0])
                bb = plsc.load_gather(x_v, [b, c0 + 1])
                idx_v[pl.ds(r * IDX_MINOR + k * LANES, LANES)] = a * VOCAB + bb
            return carry

        lax.fori_loop(0, IDX_ROWS_PER_W, id_step, 0)
        plsc.subcore_barrier()

        def gather_descs(ci, buf):
            return [
                pltpu.make_async_copy(
                    table_s.at[idx_v.at[pl.ds(
                        (ci * CHUNK_IDX_ROWS + j) * IDX_MINOR, IDX_MINOR)]],
                    buf.at[pl.ds(j * IDX_MINOR, IDX_MINOR)],
                    sem_g,
                )
                for j in range(CHUNK_IDX_ROWS)
            ]

        def out_desc(ci, buf):
            return pltpu.make_async_copy(
                buf.reshape(CHUNK_B, IDS_PER_ROW * D),
                out_hbm.at[pl.ds(b_base + ci * CHUNK_B, CHUNK_B)], sem_o)

        for d in gather_descs(0, rows_v.at[0]):
            d.start()

        def chunk_body(ci, carry):
            buf = rows_v.at[ci % 2]
            nbuf = rows_v.at[(ci + 1) % 2]

            @pl.when(ci >= 1)
            def _():
                out_desc(ci - 1, nbuf).wait()

            @pl.when(ci + 1 < N_CHUNKS)
            def _():
                for d in gather_descs(ci + 1, nbuf):
                    d.start()

            for d in gather_descs(ci, buf):
                d.wait()
            out_desc(ci, buf).start()
            return carry

        lax.fori_loop(0, N_CHUNKS, chunk_body, 0)
        out_desc(N_CHUNKS - 1, rows_v.at[(N_CHUNKS - 1) % 2]).wait()

    return sc_gather


_sc_gather = _make_sc_gather()


def kernel(X, color_table, pos_table):
    del pos_table
    ptable = jnp.concatenate(
        [jnp.repeat(color_table, VOCAB, axis=0),
         jnp.tile(color_table, (VOCAB, 1))], axis=1)
    return _sc_gather(ptable, X.astype(jnp.int32))
